# baseline (speedup 1.0000x reference)
import functools

import jax
import jax.numpy as jnp
from jax import lax
from jax.experimental import pallas as pl
from jax.experimental.pallas import tpu as pltpu
from jax.experimental.pallas import tpu_sc as plsc

D = 64
V = 100000
B = 16384
NEG = 5
K = NEG + 1
NC = 2
NS = 16
NW = NC * NS
CH = 128
NCH = B // CH
CPW = NCH // NW
L = 16
NG = CH // L
CI = K * CH

_mesh = plsc.VectorSubcoreMesh(core_axis_name="c", subcore_axis_name="s")


@functools.partial(
    pl.kernel,
    out_type=jax.ShapeDtypeStruct((NCH * K, CH), jnp.float32),
    mesh=_mesh,
    scratch_types=[
        pltpu.VMEM((CPW * K, CH), jnp.int32),
        pltpu.VMEM((CI, D), jnp.float32),
        pltpu.VMEM((CI, D), jnp.float32),
        pltpu.VMEM((CH, D), jnp.float32),
        pltpu.VMEM((CH, D), jnp.float32),
        pltpu.VMEM((K, CH), jnp.float32),
        pltpu.VMEM((K, CH), jnp.float32),
        pltpu.SemaphoreType.DMA,
        pltpu.SemaphoreType.DMA,
        pltpu.SemaphoreType.DMA,
    ],
    compiler_params=pltpu.CompilerParams(
        needs_layout_passes=False, use_tc_tiling_on_sc=False),
)
def _sc_score(idx_hbm, h_hbm, table_hbm, out_hbm,
              idx_v, rows0, rows1, h0, h1, sc0, sc1,
              sem0, sem1, sem_out):
    wid = lax.axis_index("s") * NC + lax.axis_index("c")
    lane = lax.iota(jnp.int32, L)
    rows_bufs = (rows0, rows1)
    h_bufs = (h0, h1)
    sc_bufs = (sc0, sc1)
    sems = (sem0, sem1)

    pltpu.sync_copy(idx_hbm.at[pl.ds(wid * (CPW * K), CPW * K)], idx_v)

    def fire(c):
        p = c % 2
        descs = [
            pltpu.async_copy(
                table_hbm.at[idx_v.at[c * K + k]],
                rows_bufs[p].at[pl.ds(k * CH, CH)], sems[p])
            for k in range(K)
        ]
        descs.append(
            pltpu.async_copy(
                h_hbm.at[pl.ds((wid * CPW + c) * CH, CH)],
                h_bufs[p], sems[p]))
        return descs

    pending = {0: fire(0)}
    out_descs = []
    for c in range(CPW):
        p = c % 2
        if c + 1 < CPW:
            pending[c + 1] = fire(c + 1)
        for d_ in pending.pop(c):
            d_.wait()
        rows_v, h_v, scores_v = rows_bufs[p], h_bufs[p], sc_bufs[p]
        if c >= 2:
            out_descs[c - 2].wait()

        @pl.loop(0, NG)
        def _group(g):
            b0 = g * L
            bvec = b0 + lane
            rvecs = [bvec * K + k for k in range(K)]
            zero = jnp.zeros((L,), jnp.float32)

            @pl.loop(0, D, init_carry=(lane,) + (zero,) * K, unroll=4)
            def _dstep(t, carry):
                dvec, *accs = carry
                hv = plsc.load_gather(h_v, [bvec, dvec])
                new_accs = [
                    accs[k] + hv * plsc.load_gather(rows_v, [rvecs[k], dvec])
                    for k in range(K)
                ]
                dvec = jnp.bitwise_and(dvec + 1, D - 1)
                return (dvec, *new_accs)

            accs = _dstep[1:]
            for k in range(K):
                score = 1.0 / (1.0 + jnp.exp(-accs[k]))
                scores_v[k, pl.ds(b0, L)] = score

        out_descs.append(
            pltpu.async_copy(
                scores_v, out_hbm.at[pl.ds((wid * CPW + c) * K, K)],
                sem_out))
    for d_ in out_descs[-2:]:
        d_.wait()


_NEG_CACHE = None


def _neg_flat():
    global _NEG_CACHE
    if _NEG_CACHE is None:
        neg = jax.random.randint(
            jax.random.key(123), (B, NEG), 0, V).astype(jnp.int32)
        flat = jnp.concatenate(
            [jnp.zeros((B, 1), jnp.int32), neg], axis=1).reshape(B * K)
        _NEG_CACHE = jax.block_until_ready(flat)
    return _NEG_CACHE


def kernel(h, target_index, embedding_weight):
    neg_flat = _neg_flat()
    slot = jnp.arange(B * K, dtype=jnp.int32)
    idx_flat = jnp.where(slot % K == 0,
                         jnp.take(target_index.astype(jnp.int32), slot // K),
                         neg_flat)
    out = _sc_score(idx_flat.reshape(B * K // 128, 128), h, embedding_weight)
    o = out.reshape(NCH, K, CH).transpose(1, 0, 2).reshape(K, B)
    pos_out = o[0].reshape(B, 1)
    neg_out = o[1:].T
    pos_label = jnp.ones((B, 1), dtype=jnp.float32)
    neg_label = jnp.zeros((B, NEG), dtype=jnp.float32)
    return (pos_out, pos_label, neg_out, neg_label)


# speedup vs baseline: 6.5264x; 6.5264x over previous
import functools

import jax
import jax.numpy as jnp
from jax import lax
from jax.experimental import pallas as pl
from jax.experimental.pallas import tpu as pltpu
from jax.experimental.pallas import tpu_sc as plsc

D = 64
DP = 128
V = 100000
B = 16384
NEG = 5
K = NEG + 1
NC = 2
NS = 16
NW = NC * NS
CH = 64
NCH = B // CH
CPW = NCH // NW
L = 16
NG = CH // L
CI = K * CH
NR = CI // 128
BPW = B // NW

_mesh = plsc.VectorSubcoreMesh(core_axis_name="c", subcore_axis_name="s")


@functools.partial(
    pl.kernel,
    out_type=jax.ShapeDtypeStruct((NCH * NR, 128), jnp.float32),
    mesh=_mesh,
    scratch_types=[
        pltpu.VMEM((CPW * NR, 128), jnp.int32),
        pltpu.VMEM((BPW // 128, 128), jnp.int32),
        pltpu.VMEM((CI, DP), jnp.float32),
        pltpu.VMEM((CI, DP), jnp.float32),
        pltpu.VMEM((CH, DP), jnp.float32),
        pltpu.VMEM((CH, DP), jnp.float32),
        pltpu.VMEM((NR, 128), jnp.float32),
        pltpu.VMEM((NR, 128), jnp.float32),
        pltpu.SemaphoreType.DMA,
        pltpu.SemaphoreType.DMA,
        pltpu.SemaphoreType.DMA,
    ],
    compiler_params=pltpu.CompilerParams(
        needs_layout_passes=False, use_tc_tiling_on_sc=False),
)
def _sc_score(neg_hbm, tgt_hbm, h_hbm, table_hbm, out_hbm,
              idx_v, tgt_v, rows0, rows1, h0, h1, sc0, sc1,
              sem0, sem1, sem_out):
    wid = lax.axis_index("s") * NC + lax.axis_index("c")
    lane = lax.iota(jnp.int32, L)
    rows_bufs = (rows0, rows1)
    h_bufs = (h0, h1)
    sc_bufs = (sc0, sc1)
    sems = (sem0, sem1)

    pltpu.sync_copy(neg_hbm.at[pl.ds(wid * (CPW * NR), CPW * NR)], idx_v)
    pltpu.sync_copy(tgt_hbm.at[pl.ds(wid * (BPW // 128), BPW // 128)], tgt_v)
    for b0 in range(0, BPW, L):
        tv = tgt_v[b0 // 128, pl.ds(b0 % 128, L)]
        pos = (b0 * K) + lane * K
        plsc.store_scatter(idx_v, [pos // 128, pos % 128], tv)

    def fire(c):
        p = c % 2
        descs = [
            pltpu.async_copy(
                table_hbm.at[idx_v.at[c * NR + k]],
                rows_bufs[p].at[pl.ds(k * 128, 128)], sems[p])
            for k in range(NR)
        ]
        descs.append(
            pltpu.async_copy(
                h_hbm.at[pl.ds((wid * CPW + c) * CH, CH)],
                h_bufs[p], sems[p]))
        return descs

    pending = {0: fire(0)}
    out_descs = []
    for c in range(CPW):
        p = c % 2
        if c + 1 < CPW:
            pending[c + 1] = fire(c + 1)
        for d_ in pending.pop(c):
            d_.wait()
        rows_v, h_v, scores_v = rows_bufs[p], h_bufs[p], sc_bufs[p]
        if c >= 2:
            out_descs[c - 2].wait()

        @pl.loop(0, NG)
        def _group(g):
            b0 = g * L
            bvec = b0 + lane
            rvecs = [bvec * K + k for k in range(K)]
            zero = jnp.zeros((L,), jnp.float32)

            @pl.loop(0, D, init_carry=(lane,) + (zero,) * K, unroll=4)
            def _dstep(t, carry):
                dvec, *accs = carry
                hv = plsc.load_gather(h_v, [bvec, dvec])
                new_accs = [
                    accs[k] + hv * plsc.load_gather(rows_v, [rvecs[k], dvec])
                    for k in range(K)
                ]
                dvec = jnp.bitwise_and(dvec + 1, D - 1)
                return (dvec, *new_accs)

            accs = _dstep[1:]
            for k in range(K):
                score = 1.0 / (1.0 + jnp.exp(-accs[k]))
                fp = k * CH
                scores_v[fp // 128, pl.ds(fp % 128 + b0, L)] = score

        out_descs.append(
            pltpu.async_copy(
                scores_v, out_hbm.at[pl.ds((wid * CPW + c) * NR, NR)],
                sem_out))
    for d_ in out_descs[-2:]:
        d_.wait()


_NEG_CACHE = None


def _neg_rows():
    global _NEG_CACHE
    if _NEG_CACHE is None:
        neg = jax.random.randint(
            jax.random.key(123), (B, NEG), 0, V).astype(jnp.int32)
        flat = jnp.concatenate(
            [jnp.zeros((B, 1), jnp.int32), neg], axis=1).reshape(B * K // 128,
                                                                 128)
        _NEG_CACHE = jax.block_until_ready(flat)
    return _NEG_CACHE


def kernel(h, target_index, embedding_weight):
    h_pad = jnp.pad(h, ((0, 0), (0, DP - D)))
    table_pad = jnp.pad(embedding_weight, ((0, 0), (0, DP - D)))
    tgt2d = target_index.astype(jnp.int32).reshape(B // 128, 128)
    out = _sc_score(_neg_rows(), tgt2d, h_pad, table_pad)
    o = out.reshape(NCH, K, CH).transpose(1, 0, 2).reshape(K, B)
    pos_out = o[0].reshape(B, 1)
    neg_out = o[1:].T
    pos_label = jnp.ones((B, 1), dtype=jnp.float32)
    neg_label = jnp.zeros((B, NEG), dtype=jnp.float32)
    return (pos_out, pos_label, neg_out, neg_label)


# speedup vs baseline: 7.6491x; 1.1720x over previous
import functools

import jax
import jax.numpy as jnp
from jax import lax
from jax.experimental import pallas as pl
from jax.experimental.pallas import tpu as pltpu
from jax.experimental.pallas import tpu_sc as plsc

D = 64
DP = 128
V = 100000
B = 16384
NEG = 5
K = NEG + 1
NC = 2
NS = 16
NW = NC * NS
CH = 64
NCH = B // CH
CPW = NCH // NW
L = 16
NG = CH // L
CI = K * CH
NR = CI // 128
BPW = B // NW

_mesh = plsc.VectorSubcoreMesh(core_axis_name="c", subcore_axis_name="s")


@functools.partial(
    pl.kernel,
    out_type=jax.ShapeDtypeStruct((NCH * NR, 128), jnp.float32),
    mesh=_mesh,
    scratch_types=[
        pltpu.VMEM((CPW * NR, 128), jnp.int32),
        pltpu.VMEM((BPW // 128, 128), jnp.int32),
        pltpu.VMEM((CI, DP), jnp.float32),
        pltpu.VMEM((CI, DP), jnp.float32),
        pltpu.VMEM((CH, DP), jnp.float32),
        pltpu.VMEM((CH, DP), jnp.float32),
        pltpu.VMEM((NR, 128), jnp.float32),
        pltpu.VMEM((NR, 128), jnp.float32),
        pltpu.SemaphoreType.DMA,
        pltpu.SemaphoreType.DMA,
        pltpu.SemaphoreType.DMA,
    ],
    compiler_params=pltpu.CompilerParams(
        needs_layout_passes=False, use_tc_tiling_on_sc=False),
)
def _sc_score(neg_hbm, tgt_hbm, h_hbm, table_hbm, out_hbm,
              idx_v, tgt_v, rows0, rows1, h0, h1, sc0, sc1,
              sem0, sem1, sem_out):
    wid = lax.axis_index("s") * NC + lax.axis_index("c")
    lane = lax.iota(jnp.int32, L)
    rows_bufs = (rows0, rows1)
    h_bufs = (h0, h1)
    sc_bufs = (sc0, sc1)
    sems = (sem0, sem1)

    pltpu.sync_copy(neg_hbm.at[pl.ds(wid * (CPW * NR), CPW * NR)], idx_v)
    pltpu.sync_copy(tgt_hbm.at[pl.ds(wid * (BPW // 128), BPW // 128)], tgt_v)
    for b0 in range(0, BPW, L):
        tv = tgt_v[b0 // 128, pl.ds(b0 % 128, L)]
        pos = (b0 * K) + lane * K
        plsc.store_scatter(idx_v, [pos // 128, pos % 128], tv)

    def fire(c):
        p = c % 2
        descs = [
            pltpu.async_copy(
                table_hbm.at[idx_v.at[c * NR + k]],
                rows_bufs[p].at[pl.ds(k * 128, 128)], sems[p])
            for k in range(NR)
        ]
        descs.append(
            pltpu.async_copy(
                h_hbm.at[pl.ds((wid * CPW + c) * CH, CH)],
                h_bufs[p], sems[p]))
        return descs

    pending = {0: fire(0)}
    out_descs = []
    for c in range(CPW):
        p = c % 2
        if c + 1 < CPW:
            pending[c + 1] = fire(c + 1)
        for d_ in pending.pop(c):
            d_.wait()
        rows_v, h_v, scores_v = rows_bufs[p], h_bufs[p], sc_bufs[p]
        if c >= 2:
            out_descs[c - 2].wait()

        @pl.loop(0, NG)
        def _group(g):
            b0 = g * L
            bvec = b0 + lane
            rvecs = [bvec * K + k for k in range(K)]
            zero = jnp.zeros((L,), jnp.float32)

            @pl.loop(0, D, init_carry=(lane,) + (zero,) * K, unroll=4)
            def _dstep(t, carry):
                dvec, *accs = carry
                hv = plsc.load_gather(h_v, [bvec, dvec])
                new_accs = [
                    accs[k] + hv * plsc.load_gather(rows_v, [rvecs[k], dvec])
                    for k in range(K)
                ]
                dvec = jnp.bitwise_and(dvec + 1, D - 1)
                return (dvec, *new_accs)

            accs = _dstep[1:]
            for k in range(K):
                score = 1.0 / (1.0 + jnp.exp(-accs[k]))
                fp = k * CH
                scores_v[fp // 128, pl.ds(fp % 128 + b0, L)] = score

        out_descs.append(
            pltpu.async_copy(
                scores_v, out_hbm.at[pl.ds((wid * CPW + c) * NR, NR)],
                sem_out))
    for d_ in out_descs[-2:]:
        d_.wait()


def _make_neg_rows():
    import numpy as np
    neg = jax.random.randint(
        jax.random.key(123), (B, NEG), 0, V).astype(jnp.int32)
    flat = jnp.concatenate(
        [jnp.zeros((B, 1), jnp.int32), neg], axis=1).reshape(B * K // 128,
                                                             128)
    return np.asarray(jax.block_until_ready(flat))


_NEG_ROWS = _make_neg_rows()


def _neg_rows():
    return _NEG_ROWS


def kernel(h, target_index, embedding_weight):
    h_pad = jnp.pad(h, ((0, 0), (0, DP - D)))
    table_pad = jnp.pad(embedding_weight, ((0, 0), (0, DP - D)))
    tgt2d = target_index.astype(jnp.int32).reshape(B // 128, 128)
    out = _sc_score(_neg_rows(), tgt2d, h_pad, table_pad)
    o = out.reshape(NCH, K, CH).transpose(1, 0, 2).reshape(K, B)
    pos_out = o[0].reshape(B, 1)
    neg_out = o[1:].T
    pos_label = jnp.ones((B, 1), dtype=jnp.float32)
    neg_label = jnp.zeros((B, NEG), dtype=jnp.float32)
    return (pos_out, pos_label, neg_out, neg_label)


# speedup vs baseline: 7.8459x; 1.0257x over previous
import functools

import jax
import jax.numpy as jnp
from jax import lax
from jax.experimental import pallas as pl
from jax.experimental.pallas import tpu as pltpu
from jax.experimental.pallas import tpu_sc as plsc

D = 64
DP = 128
V = 100000
B = 16384
NEG = 5
K = NEG + 1
NC = 2
NS = 16
NW = NC * NS
CH = 64
NCH = B // CH
CPW = NCH // NW
L = 16
NG = CH // L
CI = K * CH
NR = CI // 128
BPW = B // NW

_mesh = plsc.VectorSubcoreMesh(core_axis_name="c", subcore_axis_name="s")


@functools.partial(
    pl.kernel,
    out_type=jax.ShapeDtypeStruct((NCH * NR, 128), jnp.float32),
    mesh=_mesh,
    scratch_types=[
        pltpu.VMEM((CPW * NR, 128), jnp.int32),
        pltpu.VMEM((BPW // 128, 128), jnp.int32),
        pltpu.VMEM((CI, D), jnp.float32),
        pltpu.VMEM((CI, D), jnp.float32),
        pltpu.VMEM((CH, DP), jnp.float32),
        pltpu.VMEM((CH, DP), jnp.float32),
        pltpu.VMEM((NR, 128), jnp.float32),
        pltpu.VMEM((NR, 128), jnp.float32),
        pltpu.SemaphoreType.DMA,
        pltpu.SemaphoreType.DMA,
        pltpu.SemaphoreType.DMA,
    ],
    compiler_params=pltpu.CompilerParams(
        needs_layout_passes=False, use_tc_tiling_on_sc=False),
)
def _sc_score(neg_hbm, tgt_hbm, h_hbm, table_hbm, out_hbm,
              idx_v, tgt_v, rows0, rows1, h0, h1, sc0, sc1,
              sem0, sem1, sem_out):
    wid = lax.axis_index("s") * NC + lax.axis_index("c")
    lane = lax.iota(jnp.int32, L)
    rows_bufs = (rows0, rows1)
    h_bufs = (h0, h1)
    sc_bufs = (sc0, sc1)
    sems = (sem0, sem1)

    pltpu.sync_copy(neg_hbm.at[pl.ds(wid * (CPW * NR), CPW * NR)], idx_v)
    pltpu.sync_copy(tgt_hbm.at[pl.ds(wid * (BPW // 128), BPW // 128)], tgt_v)
    for b0 in range(0, BPW, L):
        tv = tgt_v[b0 // 128, pl.ds(b0 % 128, L)]
        pos = (b0 * K) + lane * K
        plsc.store_scatter(idx_v, [pos // 128, pos % 128], tv)

    def fire(c):
        p = c % 2
        descs = [
            pltpu.async_copy(
                table_hbm.at[idx_v.at[c * NR + k]],
                rows_bufs[p].at[pl.ds(k * 128, 128)], sems[p])
            for k in range(NR)
        ]
        descs.append(
            pltpu.async_copy(
                h_hbm.at[pl.ds((wid * CPW + c) * CH, CH)],
                h_bufs[p], sems[p]))
        return descs

    pending = {0: fire(0)}
    out_descs = []
    for c in range(CPW):
        p = c % 2
        if c + 1 < CPW:
            pending[c + 1] = fire(c + 1)
        for d_ in pending.pop(c):
            d_.wait()
        rows_v, h_v, scores_v = rows_bufs[p], h_bufs[p], sc_bufs[p]
        if c >= 2:
            out_descs[c - 2].wait()

        @pl.loop(0, NG)
        def _group(g):
            b0 = g * L
            bvec = b0 + lane
            rvecs = [bvec * K + k for k in range(K)]
            zero = jnp.zeros((L,), jnp.float32)

            @pl.loop(0, D, init_carry=(lane,) + (zero,) * K, unroll=4)
            def _dstep(t, carry):
                dvec, *accs = carry
                hv = plsc.load_gather(h_v, [bvec, dvec])
                new_accs = [
                    accs[k] + hv * plsc.load_gather(rows_v, [rvecs[k], dvec])
                    for k in range(K)
                ]
                dvec = jnp.bitwise_and(dvec + 1, D - 1)
                return (dvec, *new_accs)

            accs = _dstep[1:]
            for k in range(K):
                score = 1.0 / (1.0 + jnp.exp(-accs[k]))
                fp = k * CH
                scores_v[fp // 128, pl.ds(fp % 128 + b0, L)] = score

        out_descs.append(
            pltpu.async_copy(
                scores_v, out_hbm.at[pl.ds((wid * CPW + c) * NR, NR)],
                sem_out))
    for d_ in out_descs[-2:]:
        d_.wait()


def _make_neg_rows():
    import numpy as np
    neg = jax.random.randint(
        jax.random.key(123), (B, NEG), 0, V).astype(jnp.int32)
    flat = jnp.concatenate(
        [jnp.zeros((B, 1), jnp.int32), neg], axis=1).reshape(B * K // 128,
                                                             128)
    return np.asarray(jax.block_until_ready(flat))


_NEG_ROWS = _make_neg_rows()


def _neg_rows():
    return _NEG_ROWS


def kernel(h, target_index, embedding_weight):
    h_pad = jnp.pad(h, ((0, 0), (0, DP - D)))
    tgt2d = target_index.astype(jnp.int32).reshape(B // 128, 128)
    out = _sc_score(_neg_rows(), tgt2d, h_pad, embedding_weight)
    o = out.reshape(NCH, K, CH).transpose(1, 0, 2).reshape(K, B)
    pos_out = o[0].reshape(B, 1)
    neg_out = o[1:].T
    pos_label = jnp.ones((B, 1), dtype=jnp.float32)
    neg_label = jnp.zeros((B, NEG), dtype=jnp.float32)
    return (pos_out, pos_label, neg_out, neg_label)
